# Initial kernel scaffold; baseline (speedup 1.0000x reference)
import jax
import jax.numpy as jnp
from jax.experimental import pallas as pl


def kernel(x, edge_index, edge_attr, el0_W, el0_b, root0, bias0, el1_W, el1_b, root1, bias1, last_W, last_b, training_with_batch):
    raise NotImplementedError("write your pallas kernel here")


# speedup vs baseline: 1.5592x; 1.5592x over previous
import functools

import jax
import jax.numpy as jnp
from jax import lax
from jax.experimental import pallas as pl
from jax.experimental.pallas import tpu as pltpu
from jax.experimental.pallas import tpu_sc as plsc

N = 10000
E = 160000
D_FEAT = 128
N_PAD = 10240
E_PAD = 163840
EPW = E_PAD // 32
CHUNK = 128
CHUNKS_W = EPW // CHUNK
ROWS_T = N_PAD // 16


def _build_table_weights(elW, elb, in_c):
    main = jnp.moveaxis(elW.reshape(4, in_c, 8), 0, 1).reshape(in_c, 32)
    return jnp.concatenate(
        [main, elb.reshape(in_c, 8), jnp.zeros((in_c, 8), elW.dtype)], axis=1)


def _mm_body(x_ref, m_ref, o_ref):
    o_ref[...] = jnp.dot(x_ref[...], m_ref[...],
                         preferred_element_type=jnp.float32)


def _table_matmul(x_pad, M):
    k = x_pad.shape[1]
    return pl.pallas_call(
        _mm_body,
        grid=(N_PAD // 1024,),
        in_specs=[pl.BlockSpec((1024, k), lambda i: (i, 0)),
                  pl.BlockSpec((k, 48), lambda i: (0, 0))],
        out_specs=pl.BlockSpec((1024, 48), lambda i: (i, 0)),
        out_shape=jax.ShapeDtypeStruct((N_PAD, 48), jnp.float32),
    )(x_pad, M)


def _combine_body(g_ref, a_ref, o_ref):
    g = g_ref[...]
    a = a_ref[...]
    acc = a[:, 0:1] * g[:, 0:8]
    for d in range(1, 5):
        acc = acc + a[:, d:d + 1] * g[:, 8 * d:8 * d + 8]
    o_ref[...] = acc


def _combine(G, attrp):
    return pl.pallas_call(
        _combine_body,
        grid=(E_PAD // 2048,),
        in_specs=[pl.BlockSpec((2048, 48), lambda i: (i, 0)),
                  pl.BlockSpec((2048, 8), lambda i: (i, 0))],
        out_specs=pl.BlockSpec((2048, 8), lambda i: (i, 0)),
        out_shape=jax.ShapeDtypeStruct((E_PAD, 8), jnp.float32),
    )(G, attrp)


def _layer_out_body(a0_ref, a1_ref, x_ref, r_ref, b_ref, m1_ref, h_ref, q_ref):
    i = pl.program_id(0)
    rows = jax.lax.broadcasted_iota(jnp.int32, (1024, 8), 0) + i * 1024
    main = a0_ref[...] + a1_ref[...] + jnp.dot(
        x_ref[...], r_ref[...], preferred_element_type=jnp.float32)
    h = jnp.maximum(main + jnp.where(rows < N, b_ref[...], 0.0), 0.0)
    h_ref[...] = h
    q_ref[...] = jnp.dot(h, m1_ref[...], preferred_element_type=jnp.float32)


def _layer_out(agg0, agg1, x_pad, root, bias_row, M1):
    k = x_pad.shape[1]
    return pl.pallas_call(
        _layer_out_body,
        grid=(N_PAD // 1024,),
        in_specs=[pl.BlockSpec((1024, 8), lambda i: (i, 0)),
                  pl.BlockSpec((1024, 8), lambda i: (i, 0)),
                  pl.BlockSpec((1024, k), lambda i: (i, 0)),
                  pl.BlockSpec((k, 8), lambda i: (0, 0)),
                  pl.BlockSpec((1, 8), lambda i: (0, 0)),
                  pl.BlockSpec((8, 48), lambda i: (0, 0))],
        out_specs=[pl.BlockSpec((1024, 8), lambda i: (i, 0)),
                   pl.BlockSpec((1024, 48), lambda i: (i, 0))],
        out_shape=[jax.ShapeDtypeStruct((N_PAD, 8), jnp.float32),
                   jax.ShapeDtypeStruct((N_PAD, 48), jnp.float32)],
    )(agg0, agg1, x_pad, root, bias_row, M1)


def _final_body(a0_ref, a1_ref, h_ref, r_ref, b_ref, w_ref, lb_ref, o_ref):
    i = pl.program_id(0)
    rows = jax.lax.broadcasted_iota(jnp.int32, (1024, 8), 0) + i * 1024
    main = a0_ref[...] + a1_ref[...] + jnp.dot(
        h_ref[...], r_ref[...], preferred_element_type=jnp.float32)
    h2 = jnp.maximum(main + jnp.where(rows < N, b_ref[...], 0.0), 0.0)
    part = jnp.sum(h2 * w_ref[...], keepdims=True)

    @pl.when(i == 0)
    def _init():
        o_ref[...] = lb_ref[...]

    o_ref[...] += part


def _final(agg0, agg1, h, root1, bias_row, last_Wrow, last_b_row):
    return pl.pallas_call(
        _final_body,
        grid=(N_PAD // 1024,),
        in_specs=[pl.BlockSpec((1024, 8), lambda i: (i, 0)),
                  pl.BlockSpec((1024, 8), lambda i: (i, 0)),
                  pl.BlockSpec((1024, 8), lambda i: (i, 0)),
                  pl.BlockSpec((8, 8), lambda i: (0, 0)),
                  pl.BlockSpec((1, 8), lambda i: (0, 0)),
                  pl.BlockSpec((1, 8), lambda i: (0, 0)),
                  pl.BlockSpec((1, 1), lambda i: (0, 0))],
        out_specs=pl.BlockSpec((1, 1), lambda i: (0, 0)),
        out_shape=jax.ShapeDtypeStruct((1, 1), jnp.float32),
    )(agg0, agg1, h, root1, bias_row, last_Wrow, last_b_row)


def _sc_mesh():
    return plsc.VectorSubcoreMesh(core_axis_name="c", subcore_axis_name="s")


def _gather_rows(table, idx2d):
    @functools.partial(
        pl.kernel,
        mesh=_sc_mesh(),
        out_type=jax.ShapeDtypeStruct((E_PAD, 48), jnp.float32),
        compiler_params=pltpu.CompilerParams(use_tc_tiling_on_sc=False),
        scratch_types=[pltpu.VMEM((CHUNKS_W, CHUNK), jnp.int32),
                       pltpu.VMEM((CHUNK, 48), jnp.float32),
                       pltpu.SemaphoreType.DMA],
    )
    def k(table_hbm, idx_hbm, out_hbm, idx_v, rows_v, sem):
        c = lax.axis_index("c")
        s = lax.axis_index("s")
        wid = s * 2 + c
        pltpu.sync_copy(idx_hbm.at[pl.ds(wid * CHUNKS_W, CHUNKS_W)], idx_v)

        def body(j, carry):
            pltpu.async_copy(table_hbm.at[idx_v.at[j]], rows_v, sem).wait()
            pltpu.sync_copy(rows_v,
                            out_hbm.at[pl.ds(wid * EPW + j * CHUNK, CHUNK)])
            return carry

        lax.fori_loop(0, CHUNKS_W, body, 0)

    return k(table, idx2d)


def _scatter_add(msg, dst2d, zeros_n):
    @functools.partial(
        pl.kernel,
        mesh=_sc_mesh(),
        out_type=jax.ShapeDtypeStruct((2 * N_PAD, 8), jnp.float32),
        compiler_params=pltpu.CompilerParams(use_tc_tiling_on_sc=False),
        scratch_types=[pltpu.VMEM((CHUNKS_W, CHUNK), jnp.int32),
                       pltpu.VMEM((CHUNK, 8), jnp.float32),
                       pltpu.VMEM_SHARED((N_PAD, 8), jnp.float32)],
    )
    def k(msg_hbm, dst_hbm, zero_hbm, out_hbm, dst_v, mbuf, acc):
        c = lax.axis_index("c")
        s = lax.axis_index("s")
        wid = s * 2 + c
        pltpu.sync_copy(zero_hbm.at[pl.ds(s * ROWS_T, ROWS_T)],
                        acc.at[pl.ds(s * ROWS_T, ROWS_T)])
        plsc.subcore_barrier()
        pltpu.sync_copy(dst_hbm.at[pl.ds(wid * CHUNKS_W, CHUNKS_W)], dst_v)

        def body(j, carry):
            pltpu.sync_copy(msg_hbm.at[pl.ds(wid * EPW + j * CHUNK, CHUNK)],
                            mbuf)
            pltpu.sync_copy(mbuf, acc.at[dst_v.at[j]], add=True)
            return carry

        lax.fori_loop(0, CHUNKS_W, body, 0)
        plsc.subcore_barrier()
        pltpu.sync_copy(acc.at[pl.ds(s * ROWS_T, ROWS_T)],
                        out_hbm.at[pl.ds(c * N_PAD + s * ROWS_T, ROWS_T)])

    return k(msg, dst2d, zeros_n)


def kernel(x, edge_index, edge_attr, el0_W, el0_b, root0, bias0,
           el1_W, el1_b, root1, bias1, last_W, last_b,
           training_with_batch=False):
    x = x.astype(jnp.float32)
    src = edge_index[0].astype(jnp.int32)
    dst = edge_index[1].astype(jnp.int32)

    x_pad = jnp.zeros((N_PAD, D_FEAT), jnp.float32).at[:N].set(x)
    srcp = jnp.zeros((E_PAD,), jnp.int32).at[:E].set(src).reshape(-1, CHUNK)
    dstp = jnp.full((E_PAD,), N, jnp.int32).at[:E].set(dst).reshape(-1, CHUNK)
    attrp = (jnp.zeros((E_PAD, 8), jnp.float32)
             .at[:E, :4].set(edge_attr.astype(jnp.float32))
             .at[:E, 4].set(1.0))
    zeros_n = jnp.zeros((N_PAD, 8), jnp.float32)

    M0 = _build_table_weights(el0_W, el0_b, D_FEAT).astype(jnp.float32)
    M1 = _build_table_weights(el1_W, el1_b, 8).astype(jnp.float32)
    b0 = bias0.reshape(1, 8).astype(jnp.float32)
    b1 = bias1.reshape(1, 8).astype(jnp.float32)
    lb = last_b.reshape(1, 1).astype(jnp.float32)

    T0 = _table_matmul(x_pad, M0)
    G0 = _gather_rows(T0, srcp)
    msg0 = _combine(G0, attrp)
    agg0 = _scatter_add(msg0, dstp, zeros_n)
    rows_mask = jax.lax.broadcasted_iota(jnp.int32, (N_PAD, 8), 0) < N
    h, T1 = _layer_out(agg0[:N_PAD], agg0[N_PAD:], x_pad,
                       root0.astype(jnp.float32), b0, M1)
    G1 = _gather_rows(T1, srcp)
    msg1 = _combine(G1, attrp)
    agg1 = _scatter_add(msg1, dstp, zeros_n)
    out2 = _final(agg1[:N_PAD], agg1[N_PAD:], h,
                  root1.astype(jnp.float32), b1,
                  last_W.reshape(1, 8).astype(jnp.float32), lb)
    return out2.reshape(1)


# speedup vs baseline: 4.7811x; 3.0664x over previous
import functools

import jax
import jax.numpy as jnp
from jax import lax
from jax.experimental import pallas as pl
from jax.experimental.pallas import tpu as pltpu
from jax.experimental.pallas import tpu_sc as plsc

N = 10000
E = 160000
D_FEAT = 128
N_PAD = 10240
E_PAD = 163840
EPW = E_PAD // 32
CHUNK = 128
CHUNKS_W = EPW // CHUNK
ROWS_T = N_PAD // 16


def _build_table_weights(elW, elb, in_c):
    main = jnp.moveaxis(elW.reshape(4, in_c, 8), 0, 1).reshape(in_c, 32)
    return jnp.concatenate(
        [main, elb.reshape(in_c, 8), jnp.zeros((in_c, 8), elW.dtype)], axis=1)


def _mm_body(x_ref, m_ref, o_ref):
    o_ref[...] = jnp.dot(x_ref[...], m_ref[...],
                         preferred_element_type=jnp.float32)


def _table_matmul(x_pad, M):
    k = x_pad.shape[1]
    return pl.pallas_call(
        _mm_body,
        grid=(N_PAD // 1024,),
        in_specs=[pl.BlockSpec((1024, k), lambda i: (i, 0)),
                  pl.BlockSpec((k, 48), lambda i: (0, 0))],
        out_specs=pl.BlockSpec((1024, 48), lambda i: (i, 0)),
        out_shape=jax.ShapeDtypeStruct((N_PAD, 48), jnp.float32),
    )(x_pad, M)


def _layer_out_body(a0_ref, a1_ref, x_ref, r_ref, b_ref, m1_ref, h_ref, q_ref):
    i = pl.program_id(0)
    rows = jax.lax.broadcasted_iota(jnp.int32, (1024, 8), 0) + i * 1024
    main = a0_ref[...] + a1_ref[...] + jnp.dot(
        x_ref[...], r_ref[...], preferred_element_type=jnp.float32)
    h = jnp.maximum(main + jnp.where(rows < N, b_ref[...], 0.0), 0.0)
    h_ref[...] = h
    q_ref[...] = jnp.dot(h, m1_ref[...], preferred_element_type=jnp.float32)


def _layer_out(agg0, agg1, x_pad, root, bias_row, M1):
    k = x_pad.shape[1]
    return pl.pallas_call(
        _layer_out_body,
        grid=(N_PAD // 1024,),
        in_specs=[pl.BlockSpec((1024, 8), lambda i: (i, 0)),
                  pl.BlockSpec((1024, 8), lambda i: (i, 0)),
                  pl.BlockSpec((1024, k), lambda i: (i, 0)),
                  pl.BlockSpec((k, 8), lambda i: (0, 0)),
                  pl.BlockSpec((1, 8), lambda i: (0, 0)),
                  pl.BlockSpec((8, 48), lambda i: (0, 0))],
        out_specs=[pl.BlockSpec((1024, 8), lambda i: (i, 0)),
                   pl.BlockSpec((1024, 48), lambda i: (i, 0))],
        out_shape=[jax.ShapeDtypeStruct((N_PAD, 8), jnp.float32),
                   jax.ShapeDtypeStruct((N_PAD, 48), jnp.float32)],
    )(agg0, agg1, x_pad, root, bias_row, M1)


def _final_body(a0_ref, a1_ref, h_ref, r_ref, b_ref, w_ref, lb_ref, o_ref):
    i = pl.program_id(0)
    rows = jax.lax.broadcasted_iota(jnp.int32, (1024, 8), 0) + i * 1024
    main = a0_ref[...] + a1_ref[...] + jnp.dot(
        h_ref[...], r_ref[...], preferred_element_type=jnp.float32)
    h2 = jnp.maximum(main + jnp.where(rows < N, b_ref[...], 0.0), 0.0)
    part = jnp.sum(h2 * w_ref[...], keepdims=True)

    @pl.when(i == 0)
    def _init():
        o_ref[...] = lb_ref[...]

    o_ref[...] += part


def _final(agg0, agg1, h, root1, bias_row, last_Wrow, last_b_row):
    return pl.pallas_call(
        _final_body,
        grid=(N_PAD // 1024,),
        in_specs=[pl.BlockSpec((1024, 8), lambda i: (i, 0)),
                  pl.BlockSpec((1024, 8), lambda i: (i, 0)),
                  pl.BlockSpec((1024, 8), lambda i: (i, 0)),
                  pl.BlockSpec((8, 8), lambda i: (0, 0)),
                  pl.BlockSpec((1, 8), lambda i: (0, 0)),
                  pl.BlockSpec((1, 8), lambda i: (0, 0)),
                  pl.BlockSpec((1, 1), lambda i: (0, 0))],
        out_specs=pl.BlockSpec((1, 1), lambda i: (0, 0)),
        out_shape=jax.ShapeDtypeStruct((1, 1), jnp.float32),
    )(agg0, agg1, h, root1, bias_row, last_Wrow, last_b_row)


def _layer_sc(T, srcp, dstp, attr16, zeros_n):
    mesh = plsc.VectorSubcoreMesh(core_axis_name="c", subcore_axis_name="s")

    @functools.partial(
        pl.kernel,
        mesh=mesh,
        out_type=jax.ShapeDtypeStruct((2 * N_PAD, 8), jnp.float32),
        compiler_params=pltpu.CompilerParams(use_tc_tiling_on_sc=False,
                                             needs_layout_passes=False),
        scratch_types=[pltpu.VMEM((CHUNKS_W, CHUNK), jnp.int32),
                       pltpu.VMEM((CHUNKS_W, CHUNK), jnp.int32),
                       pltpu.VMEM((EPW, 8), jnp.float32),
                       pltpu.VMEM((CHUNK, 48), jnp.float32),
                       pltpu.VMEM((CHUNK, 48), jnp.float32),
                       pltpu.VMEM((CHUNK, 8), jnp.float32),
                       pltpu.VMEM_SHARED((N_PAD, 8), jnp.float32),
                       pltpu.SemaphoreType.DMA,
                       pltpu.SemaphoreType.DMA],
    )
    def k(t_hbm, src_hbm, dst_hbm, attr_hbm, zero_hbm, out_hbm,
          src_v, dst_v, attr_v, rows0, rows1, msg_v, acc, sem0, sem1):
        c = lax.axis_index("c")
        s = lax.axis_index("s")
        wid = s * 2 + c
        pltpu.sync_copy(zero_hbm.at[pl.ds(s * ROWS_T, ROWS_T)],
                        acc.at[pl.ds(s * ROWS_T, ROWS_T)])
        pltpu.sync_copy(src_hbm.at[pl.ds(wid * CHUNKS_W, CHUNKS_W)], src_v)
        pltpu.sync_copy(dst_hbm.at[pl.ds(wid * CHUNKS_W, CHUNKS_W)], dst_v)
        pltpu.sync_copy(attr_hbm.at[pl.ds(wid * EPW, EPW)], attr_v)
        plsc.subcore_barrier()

        rows = (rows0, rows1)
        sems = (sem0, sem1)
        pltpu.async_copy(t_hbm.at[src_v.at[0]], rows0, sem0)
        pltpu.async_copy(t_hbm.at[src_v.at[1]], rows1, sem1)

        iota = lax.iota(jnp.int32, 16)

        def splat(v):
            return jnp.full((16,), v, jnp.int32)

        def group(rbuf, j, g):
            rowvec = g * 16 + iota
            evec = j * CHUNK + rowvec
            cf = [plsc.load_gather(attr_v, [evec, splat(d)])
                  for d in range(4)]
            for o in range(8):
                m = plsc.load_gather(rbuf, [rowvec, splat(32 + o)])
                for d in range(4):
                    m = m + cf[d] * plsc.load_gather(
                        rbuf, [rowvec, splat(8 * d + o)])
                plsc.store_scatter(msg_v, [rowvec, splat(o)], m)

        def chunk(j, b):
            rbuf = rows[b]
            sem = sems[b]
            pltpu.make_async_copy(t_hbm.at[src_v.at[j]], rbuf, sem).wait()

            def inner(g, carry):
                group(rbuf, j, g)
                return carry

            lax.fori_loop(0, CHUNK // 16, inner, 0)

            @pl.when(j + 2 < CHUNKS_W)
            def _prefetch():
                pltpu.async_copy(t_hbm.at[src_v.at[j + 2]], rbuf, sem)

            pltpu.sync_copy(msg_v, acc.at[dst_v.at[j]], add=True)

        def outer(t, carry):
            chunk(2 * t, 0)
            chunk(2 * t + 1, 1)
            return carry

        lax.fori_loop(0, CHUNKS_W // 2, outer, 0)
        plsc.subcore_barrier()
        pltpu.sync_copy(acc.at[pl.ds(s * ROWS_T, ROWS_T)],
                        out_hbm.at[pl.ds(c * N_PAD + s * ROWS_T, ROWS_T)])

    return k(T, srcp, dstp, attr16, zeros_n)


def kernel(x, edge_index, edge_attr, el0_W, el0_b, root0, bias0,
           el1_W, el1_b, root1, bias1, last_W, last_b,
           training_with_batch=False):
    x = x.astype(jnp.float32)
    src = edge_index[0].astype(jnp.int32)
    dst = edge_index[1].astype(jnp.int32)

    x_pad = jnp.zeros((N_PAD, D_FEAT), jnp.float32).at[:N].set(x)
    srcp = jnp.full((E_PAD,), N, jnp.int32).at[:E].set(src).reshape(-1, CHUNK)
    dstp = jnp.full((E_PAD,), N, jnp.int32).at[:E].set(dst).reshape(-1, CHUNK)
    attr16 = (jnp.zeros((E_PAD, 8), jnp.float32)
              .at[:E, :4].set(edge_attr.astype(jnp.float32)))
    zeros_n = jnp.zeros((N_PAD, 8), jnp.float32)

    M0 = _build_table_weights(el0_W, el0_b, D_FEAT).astype(jnp.float32)
    M1 = _build_table_weights(el1_W, el1_b, 8).astype(jnp.float32)
    b0 = bias0.reshape(1, 8).astype(jnp.float32)
    b1 = bias1.reshape(1, 8).astype(jnp.float32)
    lb = last_b.reshape(1, 1).astype(jnp.float32)

    T0 = _table_matmul(x_pad, M0)
    agg0 = _layer_sc(T0, srcp, dstp, attr16, zeros_n)
    h, T1 = _layer_out(agg0[:N_PAD], agg0[N_PAD:], x_pad,
                       root0.astype(jnp.float32), b0, M1)
    agg1 = _layer_sc(T1, srcp, dstp, attr16, zeros_n)
    out2 = _final(agg1[:N_PAD], agg1[N_PAD:], h,
                  root1.astype(jnp.float32), b1,
                  last_W.reshape(1, 8).astype(jnp.float32), lb)
    return out2.reshape(1)


# speedup vs baseline: 4.8146x; 1.0070x over previous
import functools

import jax
import jax.numpy as jnp
from jax import lax
from jax.experimental import pallas as pl
from jax.experimental.pallas import tpu as pltpu
from jax.experimental.pallas import tpu_sc as plsc

N = 10000
E = 160000
D_FEAT = 128
N_PAD = 10240
E_PAD = 163840
EPW = E_PAD // 32
CHUNK = 128
CHUNKS_W = EPW // CHUNK
ROWS_T = N_PAD // 16


def _build_table_weights(elW, elb, in_c):
    main = jnp.moveaxis(elW.reshape(4, in_c, 8), 0, 1).reshape(in_c, 32)
    return jnp.concatenate(
        [main, elb.reshape(in_c, 8), jnp.zeros((in_c, 8), elW.dtype)], axis=1)


def _mm_body(x_ref, m_ref, o_ref):
    o_ref[...] = jnp.dot(x_ref[...], m_ref[...],
                         preferred_element_type=jnp.float32)


def _table_matmul(x_pad, M):
    k = x_pad.shape[1]
    return pl.pallas_call(
        _mm_body,
        grid=(N_PAD // 1024,),
        in_specs=[pl.BlockSpec((1024, k), lambda i: (i, 0)),
                  pl.BlockSpec((k, 48), lambda i: (0, 0))],
        out_specs=pl.BlockSpec((1024, 48), lambda i: (i, 0)),
        out_shape=jax.ShapeDtypeStruct((N_PAD, 48), jnp.float32),
    )(x_pad, M)


def _layer_out_body(a0_ref, a1_ref, x_ref, r_ref, b_ref, m1_ref, h_ref, q_ref):
    i = pl.program_id(0)
    rows = jax.lax.broadcasted_iota(jnp.int32, (1024, 8), 0) + i * 1024
    main = a0_ref[...] + a1_ref[...] + jnp.dot(
        x_ref[...], r_ref[...], preferred_element_type=jnp.float32)
    h = jnp.maximum(main + jnp.where(rows < N, b_ref[...], 0.0), 0.0)
    h_ref[...] = h
    q_ref[...] = jnp.dot(h, m1_ref[...], preferred_element_type=jnp.float32)


def _layer_out(agg0, agg1, x_pad, root, bias_row, M1):
    k = x_pad.shape[1]
    return pl.pallas_call(
        _layer_out_body,
        grid=(N_PAD // 1024,),
        in_specs=[pl.BlockSpec((1024, 8), lambda i: (i, 0)),
                  pl.BlockSpec((1024, 8), lambda i: (i, 0)),
                  pl.BlockSpec((1024, k), lambda i: (i, 0)),
                  pl.BlockSpec((k, 8), lambda i: (0, 0)),
                  pl.BlockSpec((1, 8), lambda i: (0, 0)),
                  pl.BlockSpec((8, 48), lambda i: (0, 0))],
        out_specs=[pl.BlockSpec((1024, 8), lambda i: (i, 0)),
                   pl.BlockSpec((1024, 48), lambda i: (i, 0))],
        out_shape=[jax.ShapeDtypeStruct((N_PAD, 8), jnp.float32),
                   jax.ShapeDtypeStruct((N_PAD, 48), jnp.float32)],
    )(agg0, agg1, x_pad, root, bias_row, M1)


def _final_body(a0_ref, a1_ref, h_ref, r_ref, b_ref, w_ref, lb_ref, o_ref):
    i = pl.program_id(0)
    rows = jax.lax.broadcasted_iota(jnp.int32, (1024, 8), 0) + i * 1024
    main = a0_ref[...] + a1_ref[...] + jnp.dot(
        h_ref[...], r_ref[...], preferred_element_type=jnp.float32)
    h2 = jnp.maximum(main + jnp.where(rows < N, b_ref[...], 0.0), 0.0)
    part = jnp.sum(h2 * w_ref[...], keepdims=True)

    @pl.when(i == 0)
    def _init():
        o_ref[...] = lb_ref[...]

    o_ref[...] += part


def _final(agg0, agg1, h, root1, bias_row, last_Wrow, last_b_row):
    return pl.pallas_call(
        _final_body,
        grid=(N_PAD // 1024,),
        in_specs=[pl.BlockSpec((1024, 8), lambda i: (i, 0)),
                  pl.BlockSpec((1024, 8), lambda i: (i, 0)),
                  pl.BlockSpec((1024, 8), lambda i: (i, 0)),
                  pl.BlockSpec((8, 8), lambda i: (0, 0)),
                  pl.BlockSpec((1, 8), lambda i: (0, 0)),
                  pl.BlockSpec((1, 8), lambda i: (0, 0)),
                  pl.BlockSpec((1, 1), lambda i: (0, 0))],
        out_specs=pl.BlockSpec((1, 1), lambda i: (0, 0)),
        out_shape=jax.ShapeDtypeStruct((1, 1), jnp.float32),
    )(agg0, agg1, h, root1, bias_row, last_Wrow, last_b_row)


def _layer_sc(T, srcp, dstp, attr16, zeros_n):
    mesh = plsc.VectorSubcoreMesh(core_axis_name="c", subcore_axis_name="s")

    @functools.partial(
        pl.kernel,
        mesh=mesh,
        out_type=jax.ShapeDtypeStruct((2 * N_PAD, 8), jnp.float32),
        compiler_params=pltpu.CompilerParams(use_tc_tiling_on_sc=False,
                                             needs_layout_passes=False),
        scratch_types=[pltpu.VMEM((CHUNKS_W, CHUNK), jnp.int32),
                       pltpu.VMEM((CHUNKS_W, CHUNK), jnp.int32),
                       pltpu.VMEM((EPW, 8), jnp.float32),
                       pltpu.VMEM((CHUNK, 48), jnp.float32),
                       pltpu.VMEM((CHUNK, 48), jnp.float32),
                       pltpu.VMEM((CHUNK, 48), jnp.float32),
                       pltpu.VMEM((CHUNK, 48), jnp.float32),
                       pltpu.VMEM((CHUNK, 8), jnp.float32),
                       pltpu.VMEM((CHUNK, 8), jnp.float32),
                       pltpu.VMEM_SHARED((N_PAD, 8), jnp.float32),
                       pltpu.SemaphoreType.DMA,
                       pltpu.SemaphoreType.DMA,
                       pltpu.SemaphoreType.DMA,
                       pltpu.SemaphoreType.DMA,
                       pltpu.SemaphoreType.DMA,
                       pltpu.SemaphoreType.DMA],
    )
    def k(t_hbm, src_hbm, dst_hbm, attr_hbm, zero_hbm, out_hbm,
          src_v, dst_v, attr_v, rows0, rows1, rows2, rows3, msg0, msg1,
          acc, sem0, sem1, sem2, sem3, msem0, msem1):
        c = lax.axis_index("c")
        s = lax.axis_index("s")
        wid = s * 2 + c
        pltpu.sync_copy(zero_hbm.at[pl.ds(s * ROWS_T, ROWS_T)],
                        acc.at[pl.ds(s * ROWS_T, ROWS_T)])
        pltpu.sync_copy(src_hbm.at[pl.ds(wid * CHUNKS_W, CHUNKS_W)], src_v)
        pltpu.sync_copy(dst_hbm.at[pl.ds(wid * CHUNKS_W, CHUNKS_W)], dst_v)
        pltpu.sync_copy(attr_hbm.at[pl.ds(wid * EPW, EPW)], attr_v)
        plsc.subcore_barrier()

        rows = (rows0, rows1, rows2, rows3)
        sems = (sem0, sem1, sem2, sem3)
        msgs = (msg0, msg1)
        msems = (msem0, msem1)
        for b in range(4):
            pltpu.async_copy(t_hbm.at[src_v.at[b]], rows[b], sems[b])

        iota = lax.iota(jnp.int32, 16)

        def splat(v):
            return jnp.full((16,), v, jnp.int32)

        def group(rbuf, mbuf, j, g):
            rowvec = g * 16 + iota
            evec = j * CHUNK + rowvec
            cf = [plsc.load_gather(attr_v, [evec, splat(d)])
                  for d in range(4)]
            for o in range(8):
                m = plsc.load_gather(rbuf, [rowvec, splat(32 + o)])
                for d in range(4):
                    m = m + cf[d] * plsc.load_gather(
                        rbuf, [rowvec, splat(8 * d + o)])
                plsc.store_scatter(mbuf, [rowvec, splat(o)], m)

        def chunk(j, b):
            rbuf = rows[b]
            sem = sems[b]
            mbuf = msgs[b % 2]
            msem = msems[b % 2]
            pltpu.make_async_copy(t_hbm.at[src_v.at[j]], rbuf, sem).wait()

            @pl.when(j >= 2)
            def _drain():
                pltpu.make_async_copy(mbuf, acc.at[dst_v.at[j]], msem).wait()

            def inner(g, carry):
                group(rbuf, mbuf, j, g)
                return carry

            lax.fori_loop(0, CHUNK // 16, inner, 0)

            @pl.when(j + 4 < CHUNKS_W)
            def _prefetch():
                pltpu.async_copy(t_hbm.at[src_v.at[j + 4]], rbuf, sem)

            pltpu.async_copy(mbuf, acc.at[dst_v.at[j]], msem, add=True)

        def outer(t, carry):
            for b in range(4):
                chunk(4 * t + b, b)
            return carry

        lax.fori_loop(0, CHUNKS_W // 4, outer, 0)
        for b in range(2):
            pltpu.make_async_copy(msgs[b], acc.at[dst_v.at[0]],
                                  msems[b]).wait()
        plsc.subcore_barrier()
        pltpu.sync_copy(acc.at[pl.ds(s * ROWS_T, ROWS_T)],
                        out_hbm.at[pl.ds(c * N_PAD + s * ROWS_T, ROWS_T)])

    return k(T, srcp, dstp, attr16, zeros_n)


def kernel(x, edge_index, edge_attr, el0_W, el0_b, root0, bias0,
           el1_W, el1_b, root1, bias1, last_W, last_b,
           training_with_batch=False):
    x = x.astype(jnp.float32)
    src = edge_index[0].astype(jnp.int32)
    dst = edge_index[1].astype(jnp.int32)

    x_pad = jnp.zeros((N_PAD, D_FEAT), jnp.float32).at[:N].set(x)
    srcp = jnp.full((E_PAD,), N, jnp.int32).at[:E].set(src).reshape(-1, CHUNK)
    dstp = jnp.full((E_PAD,), N, jnp.int32).at[:E].set(dst).reshape(-1, CHUNK)
    attr16 = (jnp.zeros((E_PAD, 8), jnp.float32)
              .at[:E, :4].set(edge_attr.astype(jnp.float32)))
    zeros_n = jnp.zeros((N_PAD, 8), jnp.float32)

    M0 = _build_table_weights(el0_W, el0_b, D_FEAT).astype(jnp.float32)
    M1 = _build_table_weights(el1_W, el1_b, 8).astype(jnp.float32)
    b0 = bias0.reshape(1, 8).astype(jnp.float32)
    b1 = bias1.reshape(1, 8).astype(jnp.float32)
    lb = last_b.reshape(1, 1).astype(jnp.float32)

    T0 = _table_matmul(x_pad, M0)
    agg0 = _layer_sc(T0, srcp, dstp, attr16, zeros_n)
    h, T1 = _layer_out(agg0[:N_PAD], agg0[N_PAD:], x_pad,
                       root0.astype(jnp.float32), b0, M1)
    agg1 = _layer_sc(T1, srcp, dstp, attr16, zeros_n)
    out2 = _final(agg1[:N_PAD], agg1[N_PAD:], h,
                  root1.astype(jnp.float32), b1,
                  last_W.reshape(1, 8).astype(jnp.float32), lb)
    return out2.reshape(1)


# speedup vs baseline: 4.8558x; 1.0085x over previous
import functools

import jax
import jax.numpy as jnp
from jax import lax
from jax.experimental import pallas as pl
from jax.experimental.pallas import tpu as pltpu
from jax.experimental.pallas import tpu_sc as plsc

N = 10000
E = 160000
D_FEAT = 128
N_PAD = 10240
E_PAD = 163840
EPW = E_PAD // 32
CHUNK = 128
CHUNKS_W = EPW // CHUNK
ROWS_T = N_PAD // 16


def _build_table_weights(elW, elb, in_c):
    main = jnp.moveaxis(elW.reshape(4, in_c, 8), 0, 1).reshape(in_c, 32)
    return jnp.concatenate(
        [main, elb.reshape(in_c, 8), jnp.zeros((in_c, 8), elW.dtype)], axis=1)


def _mm_body(x_ref, m_ref, o_ref):
    o_ref[...] = jnp.dot(x_ref[...], m_ref[...],
                         preferred_element_type=jnp.float32)


def _table_matmul(x_pad, M):
    k = x_pad.shape[1]
    return pl.pallas_call(
        _mm_body,
        grid=(N_PAD // 1024,),
        in_specs=[pl.BlockSpec((1024, k), lambda i: (i, 0)),
                  pl.BlockSpec((k, 48), lambda i: (0, 0))],
        out_specs=pl.BlockSpec((1024, 48), lambda i: (i, 0)),
        out_shape=jax.ShapeDtypeStruct((N_PAD, 48), jnp.float32),
    )(x_pad, M)


def _layer_out_body(a0_ref, a1_ref, x_ref, r_ref, b_ref, m1_ref, h_ref, q_ref):
    i = pl.program_id(0)
    rows = jax.lax.broadcasted_iota(jnp.int32, (1024, 8), 0) + i * 1024
    main = a0_ref[...] + a1_ref[...] + jnp.dot(
        x_ref[...], r_ref[...], preferred_element_type=jnp.float32)
    h = jnp.maximum(main + jnp.where(rows < N, b_ref[...], 0.0), 0.0)
    h_ref[...] = h
    q_ref[...] = jnp.dot(h, m1_ref[...], preferred_element_type=jnp.float32)


def _layer_out(agg0, agg1, x_pad, root, bias_row, M1):
    k = x_pad.shape[1]
    return pl.pallas_call(
        _layer_out_body,
        grid=(N_PAD // 1024,),
        in_specs=[pl.BlockSpec((1024, 8), lambda i: (i, 0)),
                  pl.BlockSpec((1024, 8), lambda i: (i, 0)),
                  pl.BlockSpec((1024, k), lambda i: (i, 0)),
                  pl.BlockSpec((k, 8), lambda i: (0, 0)),
                  pl.BlockSpec((1, 8), lambda i: (0, 0)),
                  pl.BlockSpec((8, 48), lambda i: (0, 0))],
        out_specs=[pl.BlockSpec((1024, 8), lambda i: (i, 0)),
                   pl.BlockSpec((1024, 48), lambda i: (i, 0))],
        out_shape=[jax.ShapeDtypeStruct((N_PAD, 8), jnp.float32),
                   jax.ShapeDtypeStruct((N_PAD, 48), jnp.float32)],
    )(agg0, agg1, x_pad, root, bias_row, M1)


def _final_body(a0_ref, a1_ref, h_ref, r_ref, b_ref, w_ref, lb_ref, o_ref):
    i = pl.program_id(0)
    rows = jax.lax.broadcasted_iota(jnp.int32, (1024, 8), 0) + i * 1024
    main = a0_ref[...] + a1_ref[...] + jnp.dot(
        h_ref[...], r_ref[...], preferred_element_type=jnp.float32)
    h2 = jnp.maximum(main + jnp.where(rows < N, b_ref[...], 0.0), 0.0)
    part = jnp.sum(h2 * w_ref[...], keepdims=True)

    @pl.when(i == 0)
    def _init():
        o_ref[...] = lb_ref[...]

    o_ref[...] += part


def _final(agg0, agg1, h, root1, bias_row, last_Wrow, last_b_row):
    return pl.pallas_call(
        _final_body,
        grid=(N_PAD // 1024,),
        in_specs=[pl.BlockSpec((1024, 8), lambda i: (i, 0)),
                  pl.BlockSpec((1024, 8), lambda i: (i, 0)),
                  pl.BlockSpec((1024, 8), lambda i: (i, 0)),
                  pl.BlockSpec((8, 8), lambda i: (0, 0)),
                  pl.BlockSpec((1, 8), lambda i: (0, 0)),
                  pl.BlockSpec((1, 8), lambda i: (0, 0)),
                  pl.BlockSpec((1, 1), lambda i: (0, 0))],
        out_specs=pl.BlockSpec((1, 1), lambda i: (0, 0)),
        out_shape=jax.ShapeDtypeStruct((1, 1), jnp.float32),
    )(agg0, agg1, h, root1, bias_row, last_Wrow, last_b_row)


def _layer_sc(T, srcp, dstp, attr16, zeros_n):
    mesh = plsc.VectorSubcoreMesh(core_axis_name="c", subcore_axis_name="s")

    @functools.partial(
        pl.kernel,
        mesh=mesh,
        out_type=jax.ShapeDtypeStruct((2 * N_PAD, 8), jnp.float32),
        compiler_params=pltpu.CompilerParams(use_tc_tiling_on_sc=False,
                                             needs_layout_passes=False),
        scratch_types=[pltpu.VMEM((CHUNKS_W, CHUNK), jnp.int32),
                       pltpu.VMEM((CHUNKS_W, CHUNK), jnp.int32),
                       pltpu.VMEM((EPW, 8), jnp.float32),
                       pltpu.VMEM((CHUNK, 48), jnp.float32),
                       pltpu.VMEM((CHUNK, 48), jnp.float32),
                       pltpu.VMEM((CHUNK, 48), jnp.float32),
                       pltpu.VMEM((CHUNK, 48), jnp.float32),
                       pltpu.VMEM((CHUNK, 8), jnp.float32),
                       pltpu.VMEM((CHUNK, 8), jnp.float32),
                       pltpu.VMEM_SHARED((N_PAD, 8), jnp.float32),
                       pltpu.SemaphoreType.DMA,
                       pltpu.SemaphoreType.DMA,
                       pltpu.SemaphoreType.DMA,
                       pltpu.SemaphoreType.DMA,
                       pltpu.SemaphoreType.DMA,
                       pltpu.SemaphoreType.DMA],
    )
    def k(t_hbm, src_hbm, dst_hbm, attr_hbm, zero_hbm, out_hbm,
          src_v, dst_v, attr_v, rows0, rows1, rows2, rows3, msg0, msg1,
          acc, sem0, sem1, sem2, sem3, msem0, msem1):
        c = lax.axis_index("c")
        s = lax.axis_index("s")
        wid = s * 2 + c
        pltpu.sync_copy(zero_hbm.at[pl.ds(s * ROWS_T, ROWS_T)],
                        acc.at[pl.ds(s * ROWS_T, ROWS_T)])
        pltpu.sync_copy(src_hbm.at[pl.ds(wid * CHUNKS_W, CHUNKS_W)], src_v)
        pltpu.sync_copy(dst_hbm.at[pl.ds(wid * CHUNKS_W, CHUNKS_W)], dst_v)
        pltpu.sync_copy(attr_hbm.at[pl.ds(wid * EPW, EPW)], attr_v)
        plsc.subcore_barrier()

        rows = (rows0, rows1, rows2, rows3)
        sems = (sem0, sem1, sem2, sem3)
        msgs = (msg0, msg1)
        msems = (msem0, msem1)
        for b in range(4):
            pltpu.async_copy(t_hbm.at[src_v.at[b]], rows[b], sems[b])

        iota = lax.iota(jnp.int32, 16)

        def splat(v):
            return jnp.full((16,), v, jnp.int32)

        def group(rbuf, mbuf, j, g):
            rowvec = g * 16 + iota
            evec = j * CHUNK + rowvec
            cf = [plsc.load_gather(attr_v, [evec, splat(d)])
                  for d in range(4)]
            for o in range(8):
                m = plsc.load_gather(rbuf, [rowvec, splat(32 + o)])
                for d in range(4):
                    m = m + cf[d] * plsc.load_gather(
                        rbuf, [rowvec, splat(8 * d + o)])
                plsc.store_scatter(mbuf, [rowvec, splat(o)], m)

        def chunk(j, b):
            rbuf = rows[b]
            sem = sems[b]
            mbuf = msgs[b % 2]
            msem = msems[b % 2]
            pltpu.make_async_copy(t_hbm.at[src_v.at[j]], rbuf, sem).wait()

            @pl.when(j >= 2)
            def _drain():
                pltpu.make_async_copy(mbuf, acc.at[dst_v.at[j]], msem).wait()

            @plsc.parallel_loop(0, CHUNK // 16, unroll=2)
            def _inner(g):
                group(rbuf, mbuf, j, g)

            @pl.when(j + 4 < CHUNKS_W)
            def _prefetch():
                pltpu.async_copy(t_hbm.at[src_v.at[j + 4]], rbuf, sem)

            pltpu.async_copy(mbuf, acc.at[dst_v.at[j]], msem, add=True)

        def outer(t, carry):
            for b in range(4):
                chunk(4 * t + b, b)
            return carry

        lax.fori_loop(0, CHUNKS_W // 4, outer, 0)
        for b in range(2):
            pltpu.make_async_copy(msgs[b], acc.at[dst_v.at[0]],
                                  msems[b]).wait()
        plsc.subcore_barrier()
        pltpu.sync_copy(acc.at[pl.ds(s * ROWS_T, ROWS_T)],
                        out_hbm.at[pl.ds(c * N_PAD + s * ROWS_T, ROWS_T)])

    return k(T, srcp, dstp, attr16, zeros_n)


def kernel(x, edge_index, edge_attr, el0_W, el0_b, root0, bias0,
           el1_W, el1_b, root1, bias1, last_W, last_b,
           training_with_batch=False):
    x = x.astype(jnp.float32)
    src = edge_index[0].astype(jnp.int32)
    dst = edge_index[1].astype(jnp.int32)

    x_pad = jnp.zeros((N_PAD, D_FEAT), jnp.float32).at[:N].set(x)
    srcp = jnp.full((E_PAD,), N, jnp.int32).at[:E].set(src).reshape(-1, CHUNK)
    dstp = jnp.full((E_PAD,), N, jnp.int32).at[:E].set(dst).reshape(-1, CHUNK)
    attr16 = (jnp.zeros((E_PAD, 8), jnp.float32)
              .at[:E, :4].set(edge_attr.astype(jnp.float32)))
    zeros_n = jnp.zeros((N_PAD, 8), jnp.float32)

    M0 = _build_table_weights(el0_W, el0_b, D_FEAT).astype(jnp.float32)
    M1 = _build_table_weights(el1_W, el1_b, 8).astype(jnp.float32)
    b0 = bias0.reshape(1, 8).astype(jnp.float32)
    b1 = bias1.reshape(1, 8).astype(jnp.float32)
    lb = last_b.reshape(1, 1).astype(jnp.float32)

    T0 = _table_matmul(x_pad, M0)
    agg0 = _layer_sc(T0, srcp, dstp, attr16, zeros_n)
    h, T1 = _layer_out(agg0[:N_PAD], agg0[N_PAD:], x_pad,
                       root0.astype(jnp.float32), b0, M1)
    agg1 = _layer_sc(T1, srcp, dstp, attr16, zeros_n)
    out2 = _final(agg1[:N_PAD], agg1[N_PAD:], h,
                  root1.astype(jnp.float32), b1,
                  last_W.reshape(1, 8).astype(jnp.float32), lb)
    return out2.reshape(1)


# speedup vs baseline: 4.9808x; 1.0258x over previous
import functools

import jax
import jax.numpy as jnp
from jax import lax
from jax.experimental import pallas as pl
from jax.experimental.pallas import tpu as pltpu
from jax.experimental.pallas import tpu_sc as plsc

N = 10000
E = 160000
D_FEAT = 128
N_PAD = 10240
E_PAD = 163840
EPW = E_PAD // 32
CHUNK = 128
CHUNKS_W = EPW // CHUNK
ROWS_T = N_PAD // 16


def _build_table_weights(elW, elb, in_c):
    main = jnp.moveaxis(elW.reshape(4, in_c, 8), 0, 1).reshape(in_c, 32)
    return jnp.concatenate(
        [main, elb.reshape(in_c, 8), jnp.zeros((in_c, 8), elW.dtype)], axis=1)


def _mm_body(x_ref, m_ref, o_ref):
    o_ref[...] = jnp.dot(x_ref[...], m_ref[...],
                         preferred_element_type=jnp.float32)


def _table_matmul(x_pad, M):
    k = x_pad.shape[1]
    return pl.pallas_call(
        _mm_body,
        grid=(N_PAD // 1024,),
        in_specs=[pl.BlockSpec((1024, k), lambda i: (i, 0)),
                  pl.BlockSpec((k, 48), lambda i: (0, 0))],
        out_specs=pl.BlockSpec((1024, 48), lambda i: (i, 0)),
        out_shape=jax.ShapeDtypeStruct((N_PAD, 48), jnp.float32),
    )(x_pad, M)


def _layer_out_body(a0_ref, a1_ref, x_ref, r_ref, b_ref, m1_ref, h_ref, q_ref):
    i = pl.program_id(0)
    rows = jax.lax.broadcasted_iota(jnp.int32, (1024, 8), 0) + i * 1024
    main = a0_ref[...] + a1_ref[...] + jnp.dot(
        x_ref[...], r_ref[...], preferred_element_type=jnp.float32)
    h = jnp.maximum(main + jnp.where(rows < N, b_ref[...], 0.0), 0.0)
    h_ref[...] = h
    q_ref[...] = jnp.dot(h, m1_ref[...], preferred_element_type=jnp.float32)


def _layer_out(agg, x_pad, root, bias_row, M1):
    k = x_pad.shape[1]
    return pl.pallas_call(
        _layer_out_body,
        grid=(N_PAD // 1024,),
        in_specs=[pl.BlockSpec((1024, 8), lambda i: (i, 0)),
                  pl.BlockSpec((1024, 8), lambda i: (i + N_PAD // 1024, 0)),
                  pl.BlockSpec((1024, k), lambda i: (i, 0)),
                  pl.BlockSpec((k, 8), lambda i: (0, 0)),
                  pl.BlockSpec((1, 8), lambda i: (0, 0)),
                  pl.BlockSpec((8, 48), lambda i: (0, 0))],
        out_specs=[pl.BlockSpec((1024, 8), lambda i: (i, 0)),
                   pl.BlockSpec((1024, 48), lambda i: (i, 0))],
        out_shape=[jax.ShapeDtypeStruct((N_PAD, 8), jnp.float32),
                   jax.ShapeDtypeStruct((N_PAD, 48), jnp.float32)],
    )(agg, agg, x_pad, root, bias_row, M1)


def _final_body(a0_ref, a1_ref, h_ref, r_ref, b_ref, w_ref, lb_ref, o_ref):
    i = pl.program_id(0)
    rows = jax.lax.broadcasted_iota(jnp.int32, (1024, 8), 0) + i * 1024
    main = a0_ref[...] + a1_ref[...] + jnp.dot(
        h_ref[...], r_ref[...], preferred_element_type=jnp.float32)
    h2 = jnp.maximum(main + jnp.where(rows < N, b_ref[...], 0.0), 0.0)
    part = jnp.sum(h2 * w_ref[...], keepdims=True)

    @pl.when(i == 0)
    def _init():
        o_ref[...] = lb_ref[...]

    o_ref[...] += part


def _final(agg, h, root1, bias_row, last_Wrow, last_b_row):
    return pl.pallas_call(
        _final_body,
        grid=(N_PAD // 1024,),
        in_specs=[pl.BlockSpec((1024, 8), lambda i: (i, 0)),
                  pl.BlockSpec((1024, 8), lambda i: (i + N_PAD // 1024, 0)),
                  pl.BlockSpec((1024, 8), lambda i: (i, 0)),
                  pl.BlockSpec((8, 8), lambda i: (0, 0)),
                  pl.BlockSpec((1, 8), lambda i: (0, 0)),
                  pl.BlockSpec((1, 8), lambda i: (0, 0)),
                  pl.BlockSpec((1, 1), lambda i: (0, 0))],
        out_specs=pl.BlockSpec((1, 1), lambda i: (0, 0)),
        out_shape=jax.ShapeDtypeStruct((1, 1), jnp.float32),
    )(agg, agg, h, root1, bias_row, last_Wrow, last_b_row)


def _layer_sc(T, edges2, attr16):
    mesh = plsc.VectorSubcoreMesh(core_axis_name="c", subcore_axis_name="s")

    @functools.partial(
        pl.kernel,
        mesh=mesh,
        out_type=jax.ShapeDtypeStruct((2 * N_PAD, 8), jnp.float32),
        compiler_params=pltpu.CompilerParams(use_tc_tiling_on_sc=False,
                                             needs_layout_passes=False),
        scratch_types=[pltpu.VMEM((CHUNKS_W, CHUNK), jnp.int32),
                       pltpu.VMEM((CHUNKS_W, CHUNK), jnp.int32),
                       pltpu.VMEM((EPW, 8), jnp.float32),
                       pltpu.VMEM((CHUNK, 48), jnp.float32),
                       pltpu.VMEM((CHUNK, 48), jnp.float32),
                       pltpu.VMEM((CHUNK, 48), jnp.float32),
                       pltpu.VMEM((CHUNK, 48), jnp.float32),
                       pltpu.VMEM((CHUNK, 8), jnp.float32),
                       pltpu.VMEM((CHUNK, 8), jnp.float32),
                       pltpu.VMEM_SHARED((N_PAD, 8), jnp.float32),
                       pltpu.SemaphoreType.DMA,
                       pltpu.SemaphoreType.DMA,
                       pltpu.SemaphoreType.DMA,
                       pltpu.SemaphoreType.DMA,
                       pltpu.SemaphoreType.DMA,
                       pltpu.SemaphoreType.DMA],
    )
    def k(t_hbm, ed_hbm, attr_hbm, out_hbm,
          src_v, dst_v, attr_v, rows0, rows1, rows2, rows3, msg0, msg1,
          acc, sem0, sem1, sem2, sem3, msem0, msem1):
        c = lax.axis_index("c")
        s = lax.axis_index("s")
        wid = s * 2 + c
        pltpu.sync_copy(attr_hbm.at[pl.ds(E, ROWS_T)],
                        acc.at[pl.ds(s * ROWS_T, ROWS_T)])
        pltpu.sync_copy(ed_hbm.at[0, pl.ds(wid * CHUNKS_W, CHUNKS_W)], src_v)
        pltpu.sync_copy(ed_hbm.at[1, pl.ds(wid * CHUNKS_W, CHUNKS_W)], dst_v)
        pltpu.sync_copy(attr_hbm.at[pl.ds(wid * EPW, EPW)], attr_v)
        plsc.subcore_barrier()

        rows = (rows0, rows1, rows2, rows3)
        sems = (sem0, sem1, sem2, sem3)
        msgs = (msg0, msg1)
        msems = (msem0, msem1)
        for b in range(4):
            pltpu.async_copy(t_hbm.at[src_v.at[b]], rows[b], sems[b])

        iota = lax.iota(jnp.int32, 16)

        def splat(v):
            return jnp.full((16,), v, jnp.int32)

        def group(rbuf, mbuf, j, g):
            rowvec = g * 16 + iota
            evec = j * CHUNK + rowvec
            cf = [plsc.load_gather(attr_v, [evec, splat(d)])
                  for d in range(4)]
            for o in range(8):
                m = plsc.load_gather(rbuf, [rowvec, splat(32 + o)])
                for d in range(4):
                    m = m + cf[d] * plsc.load_gather(
                        rbuf, [rowvec, splat(8 * d + o)])
                plsc.store_scatter(mbuf, [rowvec, splat(o)], m)

        def chunk(j, b):
            rbuf = rows[b]
            sem = sems[b]
            mbuf = msgs[b % 2]
            msem = msems[b % 2]
            pltpu.make_async_copy(t_hbm.at[src_v.at[j]], rbuf, sem).wait()

            @pl.when(j >= 2)
            def _drain():
                pltpu.make_async_copy(mbuf, acc.at[dst_v.at[j]], msem).wait()

            @plsc.parallel_loop(0, CHUNK // 16, unroll=2)
            def _inner(g):
                group(rbuf, mbuf, j, g)

            @pl.when(j + 4 < CHUNKS_W)
            def _prefetch():
                pltpu.async_copy(t_hbm.at[src_v.at[j + 4]], rbuf, sem)

            pltpu.async_copy(mbuf, acc.at[dst_v.at[j]], msem, add=True)

        def outer(t, carry):
            for b in range(4):
                chunk(4 * t + b, b)
            return carry

        lax.fori_loop(0, CHUNKS_W // 4, outer, 0)
        for b in range(2):
            pltpu.make_async_copy(msgs[b], acc.at[dst_v.at[0]],
                                  msems[b]).wait()
        plsc.subcore_barrier()
        pltpu.sync_copy(acc.at[pl.ds(s * ROWS_T, ROWS_T)],
                        out_hbm.at[pl.ds(c * N_PAD + s * ROWS_T, ROWS_T)])

    return k(T, edges2, attr16)


def kernel(x, edge_index, edge_attr, el0_W, el0_b, root0, bias0,
           el1_W, el1_b, root1, bias1, last_W, last_b,
           training_with_batch=False):
    x = x.astype(jnp.float32)
    src = edge_index[0].astype(jnp.int32)
    dst = edge_index[1].astype(jnp.int32)

    x_pad = jnp.zeros((N_PAD, D_FEAT), jnp.float32).at[:N].set(x)
    edges2 = (jnp.full((2, E_PAD), N, jnp.int32)
              .at[0, :E].set(src).at[1, :E].set(dst)
              .reshape(2, -1, CHUNK))
    attr16 = (jnp.zeros((E_PAD, 8), jnp.float32)
              .at[:E, :4].set(edge_attr.astype(jnp.float32)))

    M0 = _build_table_weights(el0_W, el0_b, D_FEAT).astype(jnp.float32)
    M1 = _build_table_weights(el1_W, el1_b, 8).astype(jnp.float32)
    b0 = bias0.reshape(1, 8).astype(jnp.float32)
    b1 = bias1.reshape(1, 8).astype(jnp.float32)
    lb = last_b.reshape(1, 1).astype(jnp.float32)

    T0 = _table_matmul(x_pad, M0)
    agg0 = _layer_sc(T0, edges2, attr16)
    h, T1 = _layer_out(agg0, x_pad, root0.astype(jnp.float32), b0, M1)
    agg1 = _layer_sc(T1, edges2, attr16)
    out2 = _final(agg1, h, root1.astype(jnp.float32), b1,
                  last_W.reshape(1, 8).astype(jnp.float32), lb)
    return out2.reshape(1)
